# baseline (speedup 1.0000x reference)
import functools

import jax
import jax.numpy as jnp
from jax import lax
from jax.experimental import pallas as pl
from jax.experimental.pallas import tpu as pltpu
from jax.experimental.pallas import tpu_sc as plsc

_B = 16384
_D = 16
_V1 = 1000001
_NC = 2
_NS = 16
_NW = _NC * _NS
_CHUNK = 128
_ROWS_PER_W = _B // _NW
_NCH = _ROWS_PER_W // _CHUNK

_TSUB = 64
_TL = 1024 * _TSUB
_TGRID = (_V1 + _TL - 1) // _TL
_NG = _TGRID * _TSUB * 128


def _transpose_one(x, y_ref, a_of, m_of, d, base):
    for t in range(_TSUB):
        x2 = x[:, t * 1024:(t + 1) * 1024].reshape(128, 128)
        glane = base + t * 1024 + (a_of % 8) * 128 + m_of
        x2 = jnp.where(glane < _V1, x2, 0.0)
        y_ref[t * 128:(t + 1) * 128, :] = lax.dot_general(
            x2, d, (((0,), (0,)), ((), ())),
            preferred_element_type=jnp.float32)


def _transpose_body(xw_ref, xh_ref, xh1_ref, yw_ref, yh_ref, yh1_ref):
    a_of = lax.broadcasted_iota(jnp.int32, (128, 128), 0)
    m_of = lax.broadcasted_iota(jnp.int32, (128, 128), 1)
    d = (m_of == 16 * (a_of % 8) + a_of // 8).astype(jnp.float32)
    base = pl.program_id(0) * _TL
    _transpose_one(xw_ref[...], yw_ref, a_of, m_of, d, base)
    _transpose_one(xh_ref[...], yh_ref, a_of, m_of, d, base)
    _transpose_one(xh1_ref[...], yh1_ref, a_of, m_of, d, base)


@functools.lru_cache(maxsize=None)
def _transpose_call():
    return pl.pallas_call(
        _transpose_body,
        grid=(_TGRID,),
        in_specs=[pl.BlockSpec((16, _TL), lambda i: (0, i))] * 3,
        out_specs=[pl.BlockSpec((_TSUB * 128, 128), lambda i: (i, 0))] * 3,
        out_shape=[jax.ShapeDtypeStruct((_NG, 128), jnp.float32)] * 3,
    )


def _sc_gather_body(uidx_hbm, iidx_hbm, gw_hbm, gh_hbm, gh1_hbm,
                    u_out, v_out, v1_out,
                    uidx_v, iidx_v, bufs0, bufs1, sem):
    wid = lax.axis_index("s") * _NC + lax.axis_index("c")
    base = wid * _ROWS_PER_W
    pltpu.sync_copy(uidx_hbm.at[wid], uidx_v)
    pltpu.sync_copy(iidx_hbm.at[wid], iidx_v)
    bufs = (bufs0, bufs1)
    outs = (u_out, v_out, v1_out)

    def fire(j):
        u_b, v_b, v1_b = bufs[j % 2]
        return [
            pltpu.async_copy(gw_hbm.at[uidx_v.at[j]], u_b, sem),
            pltpu.async_copy(gh_hbm.at[iidx_v.at[j]], v_b, sem),
            pltpu.async_copy(gh1_hbm.at[iidx_v.at[j]], v1_b, sem),
        ]

    def drain(j, copies):
        for c in copies:
            c.wait()
        sl = pl.ds(base + j * _CHUNK, _CHUNK)
        for buf, out in zip(bufs[j % 2], outs):
            pltpu.sync_copy(buf, out.at[sl])

    inflight = fire(0)
    for j in range(1, _NCH):
        nxt = fire(j)
        drain(j - 1, inflight)
        inflight = nxt
    drain(_NCH - 1, inflight)


@functools.lru_cache(maxsize=None)
def _sc_gather():
    row_buf = lambda: [pltpu.VMEM((_CHUNK, 128), jnp.float32) for _ in range(3)]
    return functools.partial(
        pl.kernel,
        out_type=[jax.ShapeDtypeStruct((_B, 128), jnp.float32)] * 3,
        mesh=plsc.VectorSubcoreMesh(core_axis_name="c", subcore_axis_name="s"),
        scratch_types=[
            pltpu.VMEM((_NCH, _CHUNK), jnp.int32),
            pltpu.VMEM((_NCH, _CHUNK), jnp.int32),
            row_buf(),
            row_buf(),
            pltpu.SemaphoreType.DMA,
        ],
    )(_sc_gather_body)


def _select16(raw, sub, fold):
    lane_grp = lax.broadcasted_iota(jnp.int32, raw.shape, 1) // _D
    mask = (lane_grp == sub).astype(jnp.float32)
    return jnp.dot(raw * mask, fold, preferred_element_type=jnp.float32)


def _mlp_body(drug_ref, u_ref, v_ref, v1_ref, su_ref, si_ref,
              ww1_ref, wb1_ref, ww2_ref, wb2_ref, dw1_ref, db1_ref,
              dw2_ref, g_ref, out_ref):
    fold = (lax.broadcasted_iota(jnp.int32, (128, _D), 0) % _D ==
            lax.broadcasted_iota(jnp.int32, (128, _D), 1)).astype(jnp.float32)
    u = _select16(u_ref[...], su_ref[...], fold)
    v = _select16(v_ref[...], si_ref[...], fold)
    v1 = _select16(v1_ref[...], si_ref[...], fold)
    drug = drug_ref[...]
    wh = jnp.maximum(
        jnp.dot(drug, ww1_ref[...], preferred_element_type=jnp.float32)
        + wb1_ref[...], 0.0)
    wide = (jnp.dot(wh, ww2_ref[...], preferred_element_type=jnp.float32)
            + wb2_ref[...]) * v1
    wide_t = jnp.sum(wide, axis=1, keepdims=True)
    z = jnp.concatenate([u, v], axis=1)
    h = jax.nn.sigmoid(
        jnp.dot(z, dw1_ref[...], preferred_element_type=jnp.float32)
        + db1_ref[...])
    dnn = jnp.dot(h, dw2_ref[...], preferred_element_type=jnp.float32)
    gw = g_ref[0, 0]
    gb = g_ref[0, 1]
    out_ref[...] = jax.nn.sigmoid(wide_t * gw + gb + dnn)[:, 0]


def _mlp_call(blk):
    grid = _B // blk
    full = lambda shape: pl.BlockSpec(shape, lambda i: tuple(0 for _ in shape))
    return pl.pallas_call(
        _mlp_body,
        grid=(grid,),
        in_specs=[
            pl.BlockSpec((blk, 256), lambda i: (i, 0)),
            pl.BlockSpec((blk, 128), lambda i: (i, 0)),
            pl.BlockSpec((blk, 128), lambda i: (i, 0)),
            pl.BlockSpec((blk, 128), lambda i: (i, 0)),
            pl.BlockSpec((blk, 1), lambda i: (i, 0)),
            pl.BlockSpec((blk, 1), lambda i: (i, 0)),
            full((256, 64)),
            full((1, 64)),
            full((64, _D)),
            full((1, _D)),
            full((2 * _D, _D)),
            full((1, _D)),
            full((_D, 1)),
            full((1, 2)),
        ],
        out_specs=pl.BlockSpec((blk,), lambda i: (i,)),
        out_shape=jax.ShapeDtypeStruct((_B,), jnp.float32),
    )


def kernel(x, drug_features_x, W, H, H1, wide_w1, wide_b1, wide_w2, wide_b2,
           deep_w1, deep_b1, deep_w2, g_w, g_b):
    xi = x.astype(jnp.int32)
    gw, gh, gh1 = _transpose_call()(W.T, H.T, H1.T)
    uidx = xi[:, 0]
    iidx = xi[:, 1]
    ugrp = ((uidx // 1024) * 128 + uidx % 128).reshape(_NW, _NCH, _CHUNK)
    igrp = ((iidx // 1024) * 128 + iidx % 128).reshape(_NW, _NCH, _CHUNK)
    usub = ((uidx // 128) % 8).reshape(_B, 1)
    isub = ((iidx // 128) % 8).reshape(_B, 1)
    u_raw, v_raw, v1_raw = _sc_gather()(ugrp, igrp, gw, gh, gh1)
    g = jnp.concatenate([g_w.reshape(1, 1), g_b.reshape(1, 1)], axis=1)
    out = _mlp_call(2048)(
        drug_features_x, u_raw, v_raw, v1_raw, usub, isub,
        wide_w1, wide_b1.reshape(1, 64), wide_w2, wide_b2.reshape(1, _D),
        deep_w1, deep_b1.reshape(1, _D), deep_w2, g)
    return out


# speedup vs baseline: 1.0034x; 1.0034x over previous
import functools

import jax
import jax.numpy as jnp
from jax import lax
from jax.experimental import pallas as pl
from jax.experimental.pallas import tpu as pltpu
from jax.experimental.pallas import tpu_sc as plsc

_B = 16384
_D = 16
_V1 = 1000001
_NC = 2
_NS = 16
_NW = _NC * _NS
_CHUNK = 128
_ROWS_PER_W = _B // _NW
_NCH = _ROWS_PER_W // _CHUNK

_TSUB = 32
_TL = 1024 * _TSUB
_TGRID = (_V1 + _TL - 1) // _TL
_NG = _TGRID * _TSUB * 128


def _transpose_one(x, y_ref, a_of, m_of, d, base):
    for t in range(_TSUB):
        x2 = x[:, t * 1024:(t + 1) * 1024].reshape(128, 128)
        glane = base + t * 1024 + (a_of % 8) * 128 + m_of
        x2 = jnp.where(glane < _V1, x2, 0.0)
        y_ref[t * 128:(t + 1) * 128, :] = lax.dot_general(
            x2, d, (((0,), (0,)), ((), ())),
            preferred_element_type=jnp.float32)


def _transpose_body(xw_ref, xh_ref, xh1_ref, yw_ref, yh_ref, yh1_ref):
    a_of = lax.broadcasted_iota(jnp.int32, (128, 128), 0)
    m_of = lax.broadcasted_iota(jnp.int32, (128, 128), 1)
    d = (m_of == 16 * (a_of % 8) + a_of // 8).astype(jnp.float32)
    base = pl.program_id(0) * _TL
    _transpose_one(xw_ref[...], yw_ref, a_of, m_of, d, base)
    _transpose_one(xh_ref[...], yh_ref, a_of, m_of, d, base)
    _transpose_one(xh1_ref[...], yh1_ref, a_of, m_of, d, base)


@functools.lru_cache(maxsize=None)
def _transpose_call():
    return pl.pallas_call(
        _transpose_body,
        grid=(_TGRID,),
        in_specs=[pl.BlockSpec((16, _TL), lambda i: (0, i))] * 3,
        out_specs=[pl.BlockSpec((_TSUB * 128, 128), lambda i: (i, 0))] * 3,
        out_shape=[jax.ShapeDtypeStruct((_NG, 128), jnp.float32)] * 3,
    )


def _sc_gather_body(uidx_hbm, iidx_hbm, gw_hbm, gh_hbm, gh1_hbm,
                    u_out, v_out, v1_out,
                    uidx_v, iidx_v, bufs0, bufs1, sem):
    wid = lax.axis_index("s") * _NC + lax.axis_index("c")
    base = wid * _ROWS_PER_W
    pltpu.sync_copy(uidx_hbm.at[wid], uidx_v)
    pltpu.sync_copy(iidx_hbm.at[wid], iidx_v)
    bufs = (bufs0, bufs1)
    outs = (u_out, v_out, v1_out)

    def fire(j):
        u_b, v_b, v1_b = bufs[j % 2]
        return [
            pltpu.async_copy(gw_hbm.at[uidx_v.at[j]], u_b, sem),
            pltpu.async_copy(gh_hbm.at[iidx_v.at[j]], v_b, sem),
            pltpu.async_copy(gh1_hbm.at[iidx_v.at[j]], v1_b, sem),
        ]

    def drain(j, copies):
        for c in copies:
            c.wait()
        sl = pl.ds(base + j * _CHUNK, _CHUNK)
        for buf, out in zip(bufs[j % 2], outs):
            pltpu.sync_copy(buf, out.at[sl])

    inflight = fire(0)
    for j in range(1, _NCH):
        nxt = fire(j)
        drain(j - 1, inflight)
        inflight = nxt
    drain(_NCH - 1, inflight)


@functools.lru_cache(maxsize=None)
def _sc_gather():
    row_buf = lambda: [pltpu.VMEM((_CHUNK, 128), jnp.float32) for _ in range(3)]
    return functools.partial(
        pl.kernel,
        out_type=[jax.ShapeDtypeStruct((_B, 128), jnp.float32)] * 3,
        mesh=plsc.VectorSubcoreMesh(core_axis_name="c", subcore_axis_name="s"),
        scratch_types=[
            pltpu.VMEM((_NCH, _CHUNK), jnp.int32),
            pltpu.VMEM((_NCH, _CHUNK), jnp.int32),
            row_buf(),
            row_buf(),
            pltpu.SemaphoreType.DMA,
        ],
    )(_sc_gather_body)


def _select16(raw, sub, fold):
    lane_grp = lax.broadcasted_iota(jnp.int32, raw.shape, 1) // _D
    mask = (lane_grp == sub).astype(jnp.float32)
    return jnp.dot(raw * mask, fold, preferred_element_type=jnp.float32)


def _wide_body(drug_ref, ww1_ref, wb1_ref, ww2_ref, wb2_ref, out_ref):
    wh = jnp.maximum(
        jnp.dot(drug_ref[...], ww1_ref[...], preferred_element_type=jnp.float32)
        + wb1_ref[...], 0.0)
    out_ref[...] = (jnp.dot(wh, ww2_ref[...], preferred_element_type=jnp.float32)
                    + wb2_ref[...])


def _wide_call(blk):
    grid = _B // blk
    full = lambda shape: pl.BlockSpec(shape, lambda i: tuple(0 for _ in shape))
    return pl.pallas_call(
        _wide_body,
        grid=(grid,),
        in_specs=[
            pl.BlockSpec((blk, 256), lambda i: (i, 0)),
            full((256, 64)),
            full((1, 64)),
            full((64, _D)),
            full((1, _D)),
        ],
        out_specs=pl.BlockSpec((blk, _D), lambda i: (i, 0)),
        out_shape=jax.ShapeDtypeStruct((_B, _D), jnp.float32),
    )


def _mlp_body(wp_ref, u_ref, v_ref, v1_ref, su_ref, si_ref,
              dw1_ref, db1_ref, dw2_ref, g_ref, out_ref):
    fold = (lax.broadcasted_iota(jnp.int32, (128, _D), 0) % _D ==
            lax.broadcasted_iota(jnp.int32, (128, _D), 1)).astype(jnp.float32)
    u = _select16(u_ref[...], su_ref[...], fold)
    v = _select16(v_ref[...], si_ref[...], fold)
    v1 = _select16(v1_ref[...], si_ref[...], fold)
    wide = wp_ref[...] * v1
    wide_t = jnp.sum(wide, axis=1, keepdims=True)
    z = jnp.concatenate([u, v], axis=1)
    h = jax.nn.sigmoid(
        jnp.dot(z, dw1_ref[...], preferred_element_type=jnp.float32)
        + db1_ref[...])
    dnn = jnp.dot(h, dw2_ref[...], preferred_element_type=jnp.float32)
    gw = g_ref[0, 0]
    gb = g_ref[0, 1]
    out_ref[...] = jax.nn.sigmoid(wide_t * gw + gb + dnn)[:, 0]


def _mlp_call(blk):
    grid = _B // blk
    full = lambda shape: pl.BlockSpec(shape, lambda i: tuple(0 for _ in shape))
    return pl.pallas_call(
        _mlp_body,
        grid=(grid,),
        in_specs=[
            pl.BlockSpec((blk, _D), lambda i: (i, 0)),
            pl.BlockSpec((blk, 128), lambda i: (i, 0)),
            pl.BlockSpec((blk, 128), lambda i: (i, 0)),
            pl.BlockSpec((blk, 128), lambda i: (i, 0)),
            pl.BlockSpec((blk, 1), lambda i: (i, 0)),
            pl.BlockSpec((blk, 1), lambda i: (i, 0)),
            full((2 * _D, _D)),
            full((1, _D)),
            full((_D, 1)),
            full((1, 2)),
        ],
        out_specs=pl.BlockSpec((blk,), lambda i: (i,)),
        out_shape=jax.ShapeDtypeStruct((_B,), jnp.float32),
    )


def kernel(x, drug_features_x, W, H, H1, wide_w1, wide_b1, wide_w2, wide_b2,
           deep_w1, deep_b1, deep_w2, g_w, g_b):
    xi = x.astype(jnp.int32)
    gw, gh, gh1 = _transpose_call()(W.T, H.T, H1.T)
    uidx = xi[:, 0]
    iidx = xi[:, 1]
    ugrp = ((uidx // 1024) * 128 + uidx % 128).reshape(_NW, _NCH, _CHUNK)
    igrp = ((iidx // 1024) * 128 + iidx % 128).reshape(_NW, _NCH, _CHUNK)
    usub = ((uidx // 128) % 8).reshape(_B, 1)
    isub = ((iidx // 128) % 8).reshape(_B, 1)
    u_raw, v_raw, v1_raw = _sc_gather()(ugrp, igrp, gw, gh, gh1)
    wide_pre = _wide_call(4096)(
        drug_features_x, wide_w1, wide_b1.reshape(1, 64),
        wide_w2, wide_b2.reshape(1, _D))
    g = jnp.concatenate([g_w.reshape(1, 1), g_b.reshape(1, 1)], axis=1)
    out = _mlp_call(2048)(
        wide_pre, u_raw, v_raw, v1_raw, usub, isub,
        deep_w1, deep_b1.reshape(1, _D), deep_w2, g)
    return out
